# baseline (speedup 1.0000x reference)
import jax
import jax.numpy as jnp
from jax.experimental import pallas as pl
from jax.experimental.pallas import tpu as pltpu

KNN_K = 33
SIGMA = 0.75
EPS = 1e-10
TILE = 256


def _eps_denom(x):
    return jnp.where(x < EPS, EPS, x)


def _noisy_d2(rowp_ref, rowbf_ref, allbf_ref, pts_ref):
    px = pts_ref[0, :][None, :]
    py = pts_ref[1, :][None, :]
    pz = pts_ref[2, :][None, :]
    rx = rowp_ref[0, :][:, None]
    ry = rowp_ref[1, :][:, None]
    rz = rowp_ref[2, :][:, None]
    sq_all = px * px + py * py + pz * pz
    sq_row = rx * rx + ry * ry + rz * rz
    cross = jax.lax.dot_general(
        rowbf_ref[...], allbf_ref[...],
        (((1,), (0,)), ((), ())),
        preferred_element_type=jnp.float32)
    return jnp.maximum(sq_row + sq_all - 2.0 * cross, 0.0)


def _exact_d2(rowp_ref, pts_ref):
    dx = rowp_ref[0, :][:, None] - pts_ref[0, :][None, :]
    dy = rowp_ref[1, :][:, None] - pts_ref[1, :][None, :]
    dz = rowp_ref[2, :][:, None] - pts_ref[2, :][None, :]
    return dx * dx + dy * dy + dz * dz


def _split_dot(a_f32, bhi_ref, blo_ref):
    ahi = a_f32.astype(jnp.bfloat16)
    alo = (a_f32 - ahi.astype(jnp.float32)).astype(jnp.bfloat16)
    dims = (((1,), (0,)), ((), ()))
    bhi = bhi_ref[...]
    blo = blo_ref[...]
    out = jax.lax.dot_general(ahi, bhi, dims,
                              preferred_element_type=jnp.float32)
    out += jax.lax.dot_general(ahi, blo, dims,
                               preferred_element_type=jnp.float32)
    out += jax.lax.dot_general(alo, bhi, dims,
                               preferred_element_type=jnp.float32)
    out += jax.lax.dot_general(alo, blo, dims,
                               preferred_element_type=jnp.float32)
    return out


def _stage1_kernel(pts_ref, nrm_ref, rowp_ref, rowbf_ref, allbf_ref,
                   c1hi_ref, c1lo_ref, nd4_ref, mask_ref,
                   cur_ref, kept_ref):
    d2 = _noisy_d2(rowp_ref, rowbf_ref, allbf_ref, pts_ref)
    t, p = d2.shape
    col = jax.lax.broadcasted_iota(jnp.int32, (t, p), 1)

    bits = jax.lax.bitcast_convert_type(d2, jnp.int32)

    def radix_body(b, pbits):
        cand = pbits + jax.lax.shift_left(jnp.int32(1), 30 - b)
        cnt = jnp.sum(jnp.where(bits < cand, 1.0, 0.0), axis=1,
                      keepdims=True)
        return jnp.where(cnt >= float(KNN_K), pbits, cand)

    RADIX_BITS = 16
    pbits = jax.lax.fori_loop(
        0, RADIX_BITS, radix_body, jnp.zeros((t, 1), jnp.int32))

    below = bits < pbits
    n_below = jnp.sum(jnp.where(below, 1.0, 0.0), axis=1, keepdims=True)
    kept_ref[...] = jnp.where(below, 1.0, 0.0)
    cur_ref[...] = jnp.where(below, jnp.inf, d2)

    def fill_cond(needed):
        return jnp.sum(needed) > 0.0

    def fill_body(needed):
        cur = cur_ref[...]
        m = jnp.min(cur, axis=1, keepdims=True)
        tie = cur == m
        isel = jnp.min(jnp.where(tie, col, p), axis=1, keepdims=True)
        rem = tie & (col == isel) & (needed > 0.0)
        cur_ref[...] = jnp.where(rem, jnp.inf, cur)
        kept_ref[...] = jnp.where(rem, 1.0, kept_ref[...])
        return needed - jnp.where(needed > 0.0, 1.0, 0.0)

    jax.lax.while_loop(fill_cond, fill_body, float(KNN_K) - n_below)

    kept33 = kept_ref[...] != 0.0
    m0 = jnp.min(d2, axis=1, keepdims=True)
    idrop = jnp.min(jnp.where(d2 == m0, col, p), axis=1, keepdims=True)
    kept = kept33 & (col != idrop)

    d = _exact_d2(rowp_ref, pts_ref)
    dm = jnp.where(kept, d, 0.0)
    h = jnp.sum(dm, axis=1, keepdims=True) * (4.0 / (KNN_K - 1.0))
    w = jnp.maximum(1.0 - d / _eps_denom(h), 0.0)
    w = w * w
    phi = jnp.where(kept, w * w, 0.0)
    s = _split_dot(phi, c1hi_ref, c1lo_ref)
    den = _eps_denom(s[:, 3:4])
    nd4_ref[...] = jnp.concatenate([s[:, :3] / den, h], axis=1)
    mask_ref[...] = kept.astype(jnp.int8)


def _stage2_kernel(pts_ref, ndh_ref, rowp_ref, rowndh_ref, mask_ref,
                   c2hi_ref, c2lo_ref, loss_ref):
    kept = mask_ref[...] != 0
    d = _exact_d2(rowp_ref, pts_ref)
    h = rowndh_ref[3, :][:, None]
    w = jnp.maximum(1.0 - d / _eps_denom(h), 0.0)
    w = w * w
    phi = jnp.where(kept, w * w, 0.0)

    ndx = ndh_ref[0, :][None, :]
    ndy = ndh_ref[1, :][None, :]
    ndz = ndh_ref[2, :][None, :]
    inv_all = 1.0 / jnp.maximum(jnp.sqrt(ndx * ndx + ndy * ndy + ndz * ndz),
                                1e-12)
    ux, uy, uz = ndx * inv_all, ndy * inv_all, ndz * inv_all
    s_all = ux * ux + uy * uy + uz * uz

    rdx = rowndh_ref[0, :][:, None]
    rdy = rowndh_ref[1, :][:, None]
    rdz = rowndh_ref[2, :][:, None]
    inv_row = 1.0 / jnp.maximum(jnp.sqrt(rdx * rdx + rdy * rdy + rdz * rdz),
                                1e-12)
    vx, vy, vz = rdx * inv_row, rdy * inv_row, rdz * inv_row
    s_row = vx * vx + vy * vy + vz * vz

    dot = vx * ux + vy * uy + vz * uz
    inv_sig = 1.0 / (SIGMA * SIGMA)
    normal_w = jnp.exp(-(s_row + s_all - 2.0 * dot) * inv_sig)
    w2 = phi * normal_w

    s2 = _split_dot(w2, c2hi_ref, c2lo_ref)
    rx = rowp_ref[0, :][:, None]
    ry = rowp_ref[1, :][:, None]
    rz = rowp_ref[2, :][:, None]
    num = rx * s2[:, 0:1] + ry * s2[:, 1:2] + rz * s2[:, 2:3] - s2[:, 3:4]
    den = _eps_denom(s2[:, 4:5])
    dist = num / den
    loss_ref[:] = (dist * dist)[:, 0]


def _hilo(c):
    hi = c.astype(jnp.bfloat16)
    lo = (c - hi.astype(jnp.float32)).astype(jnp.bfloat16)
    return hi, lo


@jax.jit
def _run(points, normals):
    pts = points[0].T.astype(jnp.float32)
    nrm = normals[0].T.astype(jnp.float32)
    pts_rows_bf = points[0].astype(jnp.bfloat16)
    pts_all_bf = pts.astype(jnp.bfloat16)
    p = pts.shape[1]
    grid = (p // TILE,)
    full = pl.BlockSpec((3, p), lambda i: (0, 0))
    full4 = pl.BlockSpec((4, p), lambda i: (0, 0))
    rowb = pl.BlockSpec((3, TILE), lambda i: (0, i))
    row4 = pl.BlockSpec((4, TILE), lambda i: (0, i))
    vecb = pl.BlockSpec((TILE,), lambda i: (i,))
    rowbf = pl.BlockSpec((TILE, 3), lambda i: (i, 0))
    fullbf = pl.BlockSpec((3, p), lambda i: (0, 0))
    maskb = pl.BlockSpec((TILE, p), lambda i: (i, 0))
    nd4b = pl.BlockSpec((TILE, 4), lambda i: (i, 0))
    c4b = pl.BlockSpec((p, 4), lambda i: (0, 0))
    c5b = pl.BlockSpec((p, 5), lambda i: (0, 0))

    c1 = jnp.concatenate([nrm.T, jnp.ones((p, 1), jnp.float32)], axis=1)
    c1hi, c1lo = _hilo(c1)

    nd4, mask = pl.pallas_call(
        _stage1_kernel,
        grid=grid,
        in_specs=[full, full, rowb, rowbf, fullbf, c4b, c4b],
        out_specs=[nd4b, maskb],
        out_shape=[
            jax.ShapeDtypeStruct((p, 4), jnp.float32),
            jax.ShapeDtypeStruct((p, p), jnp.int8),
        ],
        scratch_shapes=[
            pltpu.VMEM((TILE, p), jnp.float32),
            pltpu.VMEM((TILE, p), jnp.float32),
        ],
    )(pts, nrm, pts, pts_rows_bf, pts_all_bf, c1hi, c1lo)

    ndh = nd4.T
    a = jnp.sum(points[0] * nd4[:, :3], axis=1, keepdims=True)
    c2 = jnp.concatenate([nd4[:, :3], a, jnp.ones((p, 1), jnp.float32)],
                         axis=1)
    c2hi, c2lo = _hilo(c2)

    loss = pl.pallas_call(
        _stage2_kernel,
        grid=grid,
        in_specs=[full, full4, rowb, row4, maskb, c5b, c5b],
        out_specs=vecb,
        out_shape=jax.ShapeDtypeStruct((p,), jnp.float32),
    )(pts, ndh, pts, ndh, mask, c2hi, c2lo)

    return jnp.mean(loss)


def kernel(points, normals):
    return _run(points, normals)


# speedup vs baseline: 1.0133x; 1.0133x over previous
import jax
import jax.numpy as jnp
from jax.experimental import pallas as pl
from jax.experimental.pallas import tpu as pltpu

KNN_K = 33
SIGMA = 0.75
EPS = 1e-10
TILE = 256


def _eps_denom(x):
    return jnp.where(x < EPS, EPS, x)


def _noisy_d2(rowp_ref, rowbf_ref, allbf_ref, pts_ref):
    px = pts_ref[0, :][None, :]
    py = pts_ref[1, :][None, :]
    pz = pts_ref[2, :][None, :]
    rx = rowp_ref[0, :][:, None]
    ry = rowp_ref[1, :][:, None]
    rz = rowp_ref[2, :][:, None]
    sq_all = px * px + py * py + pz * pz
    sq_row = rx * rx + ry * ry + rz * rz
    cross = jax.lax.dot_general(
        rowbf_ref[...], allbf_ref[...],
        (((1,), (0,)), ((), ())),
        preferred_element_type=jnp.float32)
    return jnp.maximum(sq_row + sq_all - 2.0 * cross, 0.0)


def _exact_d2(rowp_ref, pts_ref):
    dx = rowp_ref[0, :][:, None] - pts_ref[0, :][None, :]
    dy = rowp_ref[1, :][:, None] - pts_ref[1, :][None, :]
    dz = rowp_ref[2, :][:, None] - pts_ref[2, :][None, :]
    return dx * dx + dy * dy + dz * dz


def _split_dot(a_f32, bhi_ref, blo_ref):
    ahi = a_f32.astype(jnp.bfloat16)
    alo = (a_f32 - ahi.astype(jnp.float32)).astype(jnp.bfloat16)
    dims = (((1,), (0,)), ((), ()))
    bhi = bhi_ref[...]
    blo = blo_ref[...]
    out = jax.lax.dot_general(ahi, bhi, dims,
                              preferred_element_type=jnp.float32)
    out += jax.lax.dot_general(ahi, blo, dims,
                               preferred_element_type=jnp.float32)
    out += jax.lax.dot_general(alo, bhi, dims,
                               preferred_element_type=jnp.float32)
    out += jax.lax.dot_general(alo, blo, dims,
                               preferred_element_type=jnp.float32)
    return out


def _stage1_kernel(pts_ref, nrm_ref, rowp_ref, rowbf_ref, allbf_ref,
                   c1hi_ref, c1lo_ref, nd4_ref, mask_ref,
                   cur_ref, kept_ref):
    d2 = _noisy_d2(rowp_ref, rowbf_ref, allbf_ref, pts_ref)
    t, p = d2.shape
    col = jax.lax.broadcasted_iota(jnp.int32, (t, p), 1)

    bits = jax.lax.bitcast_convert_type(d2, jnp.int32)

    def radix_body(b, carry):
        pbits, nbp = carry
        cand = pbits + jax.lax.shift_left(jnp.int32(1), 30 - b)
        cnt = jnp.sum(jnp.where(bits < cand, 1.0, 0.0), axis=1,
                      keepdims=True)
        accept = cnt >= float(KNN_K)
        return (jnp.where(accept, pbits, cand),
                jnp.where(accept, nbp, cnt))

    RADIX_BITS = 19
    pbits, n_below = jax.lax.fori_loop(
        0, RADIX_BITS, radix_body,
        (jnp.zeros((t, 1), jnp.int32), jnp.zeros((t, 1), jnp.float32)))

    below = bits < pbits
    kept_ref[...] = jnp.where(below, 1.0, 0.0)
    cur_ref[...] = jnp.where(below, jnp.inf, d2)

    def fill_cond(needed):
        return jnp.sum(needed) > 0.0

    def fill_body(needed):
        cur = cur_ref[...]
        m = jnp.min(cur, axis=1, keepdims=True)
        tie = cur == m
        isel = jnp.min(jnp.where(tie, col, p), axis=1, keepdims=True)
        rem = tie & (col == isel) & (needed > 0.0)
        cur_ref[...] = jnp.where(rem, jnp.inf, cur)
        kept_ref[...] = jnp.where(rem, 1.0, kept_ref[...])
        return needed - jnp.where(needed > 0.0, 1.0, 0.0)

    jax.lax.while_loop(fill_cond, fill_body, float(KNN_K) - n_below)

    kept33 = kept_ref[...] != 0.0
    m0 = jnp.min(d2, axis=1, keepdims=True)
    idrop = jnp.min(jnp.where(d2 == m0, col, p), axis=1, keepdims=True)
    kept = kept33 & (col != idrop)

    d = _exact_d2(rowp_ref, pts_ref)
    dm = jnp.where(kept, d, 0.0)
    h = jnp.sum(dm, axis=1, keepdims=True) * (4.0 / (KNN_K - 1.0))
    w = jnp.maximum(1.0 - d / _eps_denom(h), 0.0)
    w = w * w
    phi = jnp.where(kept, w * w, 0.0)
    s = _split_dot(phi, c1hi_ref, c1lo_ref)
    den = _eps_denom(s[:, 3:4])
    nd4_ref[...] = jnp.concatenate([s[:, :3] / den, h], axis=1)
    mask_ref[...] = kept.astype(jnp.int8)


def _stage2_kernel(pts_ref, ndh_ref, rowp_ref, rowndh_ref, mask_ref,
                   c2hi_ref, c2lo_ref, loss_ref):
    kept = mask_ref[...] != 0
    d = _exact_d2(rowp_ref, pts_ref)
    h = rowndh_ref[3, :][:, None]
    w = jnp.maximum(1.0 - d / _eps_denom(h), 0.0)
    w = w * w
    phi = jnp.where(kept, w * w, 0.0)

    ndx = ndh_ref[0, :][None, :]
    ndy = ndh_ref[1, :][None, :]
    ndz = ndh_ref[2, :][None, :]
    inv_all = 1.0 / jnp.maximum(jnp.sqrt(ndx * ndx + ndy * ndy + ndz * ndz),
                                1e-12)
    ux, uy, uz = ndx * inv_all, ndy * inv_all, ndz * inv_all
    s_all = ux * ux + uy * uy + uz * uz

    rdx = rowndh_ref[0, :][:, None]
    rdy = rowndh_ref[1, :][:, None]
    rdz = rowndh_ref[2, :][:, None]
    inv_row = 1.0 / jnp.maximum(jnp.sqrt(rdx * rdx + rdy * rdy + rdz * rdz),
                                1e-12)
    vx, vy, vz = rdx * inv_row, rdy * inv_row, rdz * inv_row
    s_row = vx * vx + vy * vy + vz * vz

    dot = vx * ux + vy * uy + vz * uz
    inv_sig = 1.0 / (SIGMA * SIGMA)
    normal_w = jnp.exp(-(s_row + s_all - 2.0 * dot) * inv_sig)
    w2 = phi * normal_w

    s2 = _split_dot(w2, c2hi_ref, c2lo_ref)
    rx = rowp_ref[0, :][:, None]
    ry = rowp_ref[1, :][:, None]
    rz = rowp_ref[2, :][:, None]
    num = rx * s2[:, 0:1] + ry * s2[:, 1:2] + rz * s2[:, 2:3] - s2[:, 3:4]
    den = _eps_denom(s2[:, 4:5])
    dist = num / den
    loss_ref[:] = (dist * dist)[:, 0]


def _hilo(c):
    hi = c.astype(jnp.bfloat16)
    lo = (c - hi.astype(jnp.float32)).astype(jnp.bfloat16)
    return hi, lo


@jax.jit
def _run(points, normals):
    pts = points[0].T.astype(jnp.float32)
    nrm = normals[0].T.astype(jnp.float32)
    pts_rows_bf = points[0].astype(jnp.bfloat16)
    pts_all_bf = pts.astype(jnp.bfloat16)
    p = pts.shape[1]
    grid = (p // TILE,)
    full = pl.BlockSpec((3, p), lambda i: (0, 0))
    full4 = pl.BlockSpec((4, p), lambda i: (0, 0))
    rowb = pl.BlockSpec((3, TILE), lambda i: (0, i))
    row4 = pl.BlockSpec((4, TILE), lambda i: (0, i))
    vecb = pl.BlockSpec((TILE,), lambda i: (i,))
    rowbf = pl.BlockSpec((TILE, 3), lambda i: (i, 0))
    fullbf = pl.BlockSpec((3, p), lambda i: (0, 0))
    maskb = pl.BlockSpec((TILE, p), lambda i: (i, 0))
    nd4b = pl.BlockSpec((TILE, 4), lambda i: (i, 0))
    c4b = pl.BlockSpec((p, 4), lambda i: (0, 0))
    c5b = pl.BlockSpec((p, 5), lambda i: (0, 0))

    c1 = jnp.concatenate([nrm.T, jnp.ones((p, 1), jnp.float32)], axis=1)
    c1hi, c1lo = _hilo(c1)

    nd4, mask = pl.pallas_call(
        _stage1_kernel,
        grid=grid,
        in_specs=[full, full, rowb, rowbf, fullbf, c4b, c4b],
        out_specs=[nd4b, maskb],
        out_shape=[
            jax.ShapeDtypeStruct((p, 4), jnp.float32),
            jax.ShapeDtypeStruct((p, p), jnp.int8),
        ],
        scratch_shapes=[
            pltpu.VMEM((TILE, p), jnp.float32),
            pltpu.VMEM((TILE, p), jnp.float32),
        ],
    )(pts, nrm, pts, pts_rows_bf, pts_all_bf, c1hi, c1lo)

    ndh = nd4.T
    a = jnp.sum(points[0] * nd4[:, :3], axis=1, keepdims=True)
    c2 = jnp.concatenate([nd4[:, :3], a, jnp.ones((p, 1), jnp.float32)],
                         axis=1)
    c2hi, c2lo = _hilo(c2)

    loss = pl.pallas_call(
        _stage2_kernel,
        grid=grid,
        in_specs=[full, full4, rowb, row4, maskb, c5b, c5b],
        out_specs=vecb,
        out_shape=jax.ShapeDtypeStruct((p,), jnp.float32),
    )(pts, ndh, pts, ndh, mask, c2hi, c2lo)

    return jnp.mean(loss)


def kernel(points, normals):
    return _run(points, normals)


# speedup vs baseline: 1.0265x; 1.0130x over previous
import jax
import jax.numpy as jnp
from jax.experimental import pallas as pl
from jax.experimental.pallas import tpu as pltpu

KNN_K = 33
SIGMA = 0.75
EPS = 1e-10
TILE = 256


def _eps_denom(x):
    return jnp.where(x < EPS, EPS, x)


def _noisy_d2(rowp_ref, rowbf_ref, allbf_ref, pts_ref):
    px = pts_ref[0, :][None, :]
    py = pts_ref[1, :][None, :]
    pz = pts_ref[2, :][None, :]
    rx = rowp_ref[0, :][:, None]
    ry = rowp_ref[1, :][:, None]
    rz = rowp_ref[2, :][:, None]
    sq_all = px * px + py * py + pz * pz
    sq_row = rx * rx + ry * ry + rz * rz
    cross = jax.lax.dot_general(
        rowbf_ref[...], allbf_ref[...],
        (((1,), (0,)), ((), ())),
        preferred_element_type=jnp.float32)
    return jnp.maximum(sq_row + sq_all - 2.0 * cross, 0.0)


def _exact_d2(rowp_ref, pts_ref):
    dx = rowp_ref[0, :][:, None] - pts_ref[0, :][None, :]
    dy = rowp_ref[1, :][:, None] - pts_ref[1, :][None, :]
    dz = rowp_ref[2, :][:, None] - pts_ref[2, :][None, :]
    return dx * dx + dy * dy + dz * dz


def _split_dot(a_f32, bhi_ref, blo_ref):
    ahi = a_f32.astype(jnp.bfloat16)
    alo = (a_f32 - ahi.astype(jnp.float32)).astype(jnp.bfloat16)
    dims = (((1,), (0,)), ((), ()))
    bhi = bhi_ref[...]
    blo = blo_ref[...]
    out = jax.lax.dot_general(ahi, bhi, dims,
                              preferred_element_type=jnp.float32)
    out += jax.lax.dot_general(ahi, blo, dims,
                               preferred_element_type=jnp.float32)
    out += jax.lax.dot_general(alo, bhi, dims,
                               preferred_element_type=jnp.float32)
    out += jax.lax.dot_general(alo, blo, dims,
                               preferred_element_type=jnp.float32)
    return out


def _stage1_kernel(pts_ref, nrm_ref, rowp_ref, rowbf_ref, allbf_ref,
                   c1hi_ref, c1lo_ref, nd4_ref, mask_ref,
                   cur_ref, kept_ref):
    d2 = _noisy_d2(rowp_ref, rowbf_ref, allbf_ref, pts_ref)
    t, p = d2.shape
    col = jax.lax.broadcasted_iota(jnp.int32, (t, p), 1)

    bits = jax.lax.bitcast_convert_type(d2, jnp.int32)

    def radix_body(b, carry):
        pbits, nbp = carry
        cand = pbits + jax.lax.shift_left(jnp.int32(1), 30 - b)
        ind = jax.lax.shift_right_logical(bits - cand, 31)
        cnt = jnp.sum(ind, axis=1, keepdims=True)
        accept = cnt >= KNN_K
        return (jnp.where(accept, pbits, cand),
                jnp.where(accept, nbp, cnt))

    RADIX_BITS = 19
    pbits, n_below = jax.lax.fori_loop(
        0, RADIX_BITS, radix_body,
        (jnp.zeros((t, 1), jnp.int32), jnp.zeros((t, 1), jnp.int32)))

    below = bits < pbits
    kept_ref[...] = jnp.where(below, 1.0, 0.0)
    cur_ref[...] = jnp.where(below, jnp.inf, d2)

    def fill_cond(needed):
        return jnp.sum(needed) > 0

    def fill_body(needed):
        cur = cur_ref[...]
        m = jnp.min(cur, axis=1, keepdims=True)
        tie_idx = jnp.where(cur == m, col, p)
        isel = jnp.min(tie_idx, axis=1, keepdims=True)
        rem = (tie_idx == isel) & (needed > 0)
        cur_ref[...] = jnp.where(rem, jnp.inf, cur)
        kept_ref[...] = jnp.where(rem, 1.0, kept_ref[...])
        return needed - jnp.where(needed > 0, 1, 0)

    jax.lax.while_loop(fill_cond, fill_body, KNN_K - n_below)

    kept33 = kept_ref[...] != 0.0
    m0 = jnp.min(d2, axis=1, keepdims=True)
    idrop = jnp.min(jnp.where(d2 == m0, col, p), axis=1, keepdims=True)
    kept = kept33 & (col != idrop)

    d = _exact_d2(rowp_ref, pts_ref)
    dm = jnp.where(kept, d, 0.0)
    h = jnp.sum(dm, axis=1, keepdims=True) * (4.0 / (KNN_K - 1.0))
    w = jnp.maximum(1.0 - d / _eps_denom(h), 0.0)
    w = w * w
    phi = jnp.where(kept, w * w, 0.0)
    s = _split_dot(phi, c1hi_ref, c1lo_ref)
    den = _eps_denom(s[:, 3:4])
    nd4_ref[...] = jnp.concatenate([s[:, :3] / den, h], axis=1)
    mask_ref[...] = kept.astype(jnp.int8)


def _stage2_kernel(pts_ref, ndh_ref, rowp_ref, rowndh_ref, mask_ref,
                   c2hi_ref, c2lo_ref, loss_ref):
    kept = mask_ref[...] != 0
    d = _exact_d2(rowp_ref, pts_ref)
    h = rowndh_ref[3, :][:, None]
    w = jnp.maximum(1.0 - d / _eps_denom(h), 0.0)
    w = w * w
    phi = jnp.where(kept, w * w, 0.0)

    ndx = ndh_ref[0, :][None, :]
    ndy = ndh_ref[1, :][None, :]
    ndz = ndh_ref[2, :][None, :]
    inv_all = 1.0 / jnp.maximum(jnp.sqrt(ndx * ndx + ndy * ndy + ndz * ndz),
                                1e-12)
    ux, uy, uz = ndx * inv_all, ndy * inv_all, ndz * inv_all
    s_all = ux * ux + uy * uy + uz * uz

    rdx = rowndh_ref[0, :][:, None]
    rdy = rowndh_ref[1, :][:, None]
    rdz = rowndh_ref[2, :][:, None]
    inv_row = 1.0 / jnp.maximum(jnp.sqrt(rdx * rdx + rdy * rdy + rdz * rdz),
                                1e-12)
    vx, vy, vz = rdx * inv_row, rdy * inv_row, rdz * inv_row
    s_row = vx * vx + vy * vy + vz * vz

    dot = vx * ux + vy * uy + vz * uz
    inv_sig = 1.0 / (SIGMA * SIGMA)
    normal_w = jnp.exp(-(s_row + s_all - 2.0 * dot) * inv_sig)
    w2 = phi * normal_w

    s2 = _split_dot(w2, c2hi_ref, c2lo_ref)
    rx = rowp_ref[0, :][:, None]
    ry = rowp_ref[1, :][:, None]
    rz = rowp_ref[2, :][:, None]
    num = rx * s2[:, 0:1] + ry * s2[:, 1:2] + rz * s2[:, 2:3] - s2[:, 3:4]
    den = _eps_denom(s2[:, 4:5])
    dist = num / den
    loss_ref[:] = (dist * dist)[:, 0]


def _hilo(c):
    hi = c.astype(jnp.bfloat16)
    lo = (c - hi.astype(jnp.float32)).astype(jnp.bfloat16)
    return hi, lo


@jax.jit
def _run(points, normals):
    pts = points[0].T.astype(jnp.float32)
    nrm = normals[0].T.astype(jnp.float32)
    pts_rows_bf = points[0].astype(jnp.bfloat16)
    pts_all_bf = pts.astype(jnp.bfloat16)
    p = pts.shape[1]
    grid = (p // TILE,)
    full = pl.BlockSpec((3, p), lambda i: (0, 0))
    full4 = pl.BlockSpec((4, p), lambda i: (0, 0))
    rowb = pl.BlockSpec((3, TILE), lambda i: (0, i))
    row4 = pl.BlockSpec((4, TILE), lambda i: (0, i))
    vecb = pl.BlockSpec((TILE,), lambda i: (i,))
    rowbf = pl.BlockSpec((TILE, 3), lambda i: (i, 0))
    fullbf = pl.BlockSpec((3, p), lambda i: (0, 0))
    maskb = pl.BlockSpec((TILE, p), lambda i: (i, 0))
    nd4b = pl.BlockSpec((TILE, 4), lambda i: (i, 0))
    c4b = pl.BlockSpec((p, 4), lambda i: (0, 0))
    c5b = pl.BlockSpec((p, 5), lambda i: (0, 0))

    c1 = jnp.concatenate([nrm.T, jnp.ones((p, 1), jnp.float32)], axis=1)
    c1hi, c1lo = _hilo(c1)

    nd4, mask = pl.pallas_call(
        _stage1_kernel,
        grid=grid,
        in_specs=[full, full, rowb, rowbf, fullbf, c4b, c4b],
        out_specs=[nd4b, maskb],
        out_shape=[
            jax.ShapeDtypeStruct((p, 4), jnp.float32),
            jax.ShapeDtypeStruct((p, p), jnp.int8),
        ],
        scratch_shapes=[
            pltpu.VMEM((TILE, p), jnp.float32),
            pltpu.VMEM((TILE, p), jnp.float32),
        ],
    )(pts, nrm, pts, pts_rows_bf, pts_all_bf, c1hi, c1lo)

    ndh = nd4.T
    a = jnp.sum(points[0] * nd4[:, :3], axis=1, keepdims=True)
    c2 = jnp.concatenate([nd4[:, :3], a, jnp.ones((p, 1), jnp.float32)],
                         axis=1)
    c2hi, c2lo = _hilo(c2)

    loss = pl.pallas_call(
        _stage2_kernel,
        grid=grid,
        in_specs=[full, full4, rowb, row4, maskb, c5b, c5b],
        out_specs=vecb,
        out_shape=jax.ShapeDtypeStruct((p,), jnp.float32),
    )(pts, ndh, pts, ndh, mask, c2hi, c2lo)

    return jnp.mean(loss)


def kernel(points, normals):
    return _run(points, normals)


# speedup vs baseline: 1.0286x; 1.0021x over previous
import jax
import jax.numpy as jnp
from jax.experimental import pallas as pl
from jax.experimental.pallas import tpu as pltpu

KNN_K = 33
SIGMA = 0.75
EPS = 1e-10
TILE = 256


def _eps_denom(x):
    return jnp.where(x < EPS, EPS, x)


def _noisy_d2(rowp_ref, rowbf_ref, allbf_ref, pts_ref):
    px = pts_ref[0, :][None, :]
    py = pts_ref[1, :][None, :]
    pz = pts_ref[2, :][None, :]
    rx = rowp_ref[0, :][:, None]
    ry = rowp_ref[1, :][:, None]
    rz = rowp_ref[2, :][:, None]
    sq_all = px * px + py * py + pz * pz
    sq_row = rx * rx + ry * ry + rz * rz
    cross = jax.lax.dot_general(
        rowbf_ref[...], allbf_ref[...],
        (((1,), (0,)), ((), ())),
        preferred_element_type=jnp.float32)
    return jnp.maximum(sq_row + sq_all - 2.0 * cross, 0.0)


def _exact_d2(rowp_ref, pts_ref):
    dx = rowp_ref[0, :][:, None] - pts_ref[0, :][None, :]
    dy = rowp_ref[1, :][:, None] - pts_ref[1, :][None, :]
    dz = rowp_ref[2, :][:, None] - pts_ref[2, :][None, :]
    return dx * dx + dy * dy + dz * dz


def _split_dot(a_f32, bhi_ref, blo_ref):
    ahi = a_f32.astype(jnp.bfloat16)
    alo = (a_f32 - ahi.astype(jnp.float32)).astype(jnp.bfloat16)
    dims = (((1,), (0,)), ((), ()))
    bhi = bhi_ref[...]
    blo = blo_ref[...]
    out = jax.lax.dot_general(ahi, bhi, dims,
                              preferred_element_type=jnp.float32)
    out += jax.lax.dot_general(ahi, blo, dims,
                               preferred_element_type=jnp.float32)
    out += jax.lax.dot_general(alo, bhi, dims,
                               preferred_element_type=jnp.float32)
    out += jax.lax.dot_general(alo, blo, dims,
                               preferred_element_type=jnp.float32)
    return out


def _stage1_kernel(pts_ref, rowp_ref, rowbf_ref, allbf_ref,
                   c1hi_ref, c1lo_ref, nd4_ref, mask_ref,
                   cur_ref, kept_ref):
    d2 = _noisy_d2(rowp_ref, rowbf_ref, allbf_ref, pts_ref)
    t, p = d2.shape
    col = jax.lax.broadcasted_iota(jnp.int32, (t, p), 1)

    bits = jax.lax.bitcast_convert_type(d2, jnp.int32)

    def radix_body(b, carry):
        pbits, nbp = carry
        cand = pbits + jax.lax.shift_left(jnp.int32(1), 30 - b)
        ind = jax.lax.shift_right_logical(bits - cand, 31)
        cnt = jnp.sum(ind, axis=1, keepdims=True)
        accept = cnt >= KNN_K
        return (jnp.where(accept, pbits, cand),
                jnp.where(accept, nbp, cnt))

    RADIX_BITS = 19
    pbits, n_below = jax.lax.fori_loop(
        0, RADIX_BITS, radix_body,
        (jnp.zeros((t, 1), jnp.int32), jnp.zeros((t, 1), jnp.int32)))

    below = bits < pbits
    kept_ref[...] = jnp.where(below, 1.0, 0.0)
    cur_ref[...] = jnp.where(below, jnp.inf, d2)

    def fill_cond(needed):
        return jnp.sum(needed) > 0

    def fill_body(needed):
        cur = cur_ref[...]
        m = jnp.min(cur, axis=1, keepdims=True)
        tie_idx = jnp.where(cur == m, col, p)
        isel = jnp.min(tie_idx, axis=1, keepdims=True)
        rem = (tie_idx == isel) & (needed > 0)
        cur_ref[...] = jnp.where(rem, jnp.inf, cur)
        kept_ref[...] = jnp.where(rem, 1.0, kept_ref[...])
        return needed - jnp.where(needed > 0, 1, 0)

    jax.lax.while_loop(fill_cond, fill_body, KNN_K - n_below)

    kept33 = kept_ref[...] != 0.0
    m0 = jnp.min(d2, axis=1, keepdims=True)
    idrop = jnp.min(jnp.where(d2 == m0, col, p), axis=1, keepdims=True)
    kept = kept33 & (col != idrop)

    d = _exact_d2(rowp_ref, pts_ref)
    dm = jnp.where(kept, d, 0.0)
    h = jnp.sum(dm, axis=1, keepdims=True) * (4.0 / (KNN_K - 1.0))
    w = jnp.maximum(1.0 - d / _eps_denom(h), 0.0)
    w = w * w
    phi = jnp.where(kept, w * w, 0.0)
    s = _split_dot(phi, c1hi_ref, c1lo_ref)
    den = _eps_denom(s[:, 3:4])
    nd4_ref[...] = jnp.concatenate([s[:, :3] / den, h], axis=1)
    mask_ref[...] = kept.astype(jnp.int8)


def _stage2_kernel(pts_ref, ndh_ref, rowp_ref, rowndh_ref, mask_ref,
                   c2hi_ref, c2lo_ref, loss_ref):
    kept = mask_ref[...] != 0
    d = _exact_d2(rowp_ref, pts_ref)
    h = rowndh_ref[3, :][:, None]
    w = jnp.maximum(1.0 - d / _eps_denom(h), 0.0)
    w = w * w
    phi = jnp.where(kept, w * w, 0.0)

    ndx = ndh_ref[0, :][None, :]
    ndy = ndh_ref[1, :][None, :]
    ndz = ndh_ref[2, :][None, :]
    inv_all = 1.0 / jnp.maximum(jnp.sqrt(ndx * ndx + ndy * ndy + ndz * ndz),
                                1e-12)
    ux, uy, uz = ndx * inv_all, ndy * inv_all, ndz * inv_all
    s_all = ux * ux + uy * uy + uz * uz

    rdx = rowndh_ref[0, :][:, None]
    rdy = rowndh_ref[1, :][:, None]
    rdz = rowndh_ref[2, :][:, None]
    inv_row = 1.0 / jnp.maximum(jnp.sqrt(rdx * rdx + rdy * rdy + rdz * rdz),
                                1e-12)
    vx, vy, vz = rdx * inv_row, rdy * inv_row, rdz * inv_row
    s_row = vx * vx + vy * vy + vz * vz

    dot = vx * ux + vy * uy + vz * uz
    inv_sig = 1.0 / (SIGMA * SIGMA)
    normal_w = jnp.exp(-(s_row + s_all - 2.0 * dot) * inv_sig)
    w2 = phi * normal_w

    s2 = _split_dot(w2, c2hi_ref, c2lo_ref)
    rx = rowp_ref[0, :][:, None]
    ry = rowp_ref[1, :][:, None]
    rz = rowp_ref[2, :][:, None]
    num = rx * s2[:, 0:1] + ry * s2[:, 1:2] + rz * s2[:, 2:3] - s2[:, 3:4]
    den = _eps_denom(s2[:, 4:5])
    dist = num / den
    loss_ref[:] = (dist * dist)[:, 0]


def _hilo(c):
    hi = c.astype(jnp.bfloat16)
    lo = (c - hi.astype(jnp.float32)).astype(jnp.bfloat16)
    return hi, lo


@jax.jit
def _run(points, normals):
    pts = points[0].T.astype(jnp.float32)
    nrm = normals[0].T.astype(jnp.float32)
    pts_rows_bf = points[0].astype(jnp.bfloat16)
    pts_all_bf = pts.astype(jnp.bfloat16)
    p = pts.shape[1]
    grid = (p // TILE,)
    full = pl.BlockSpec((3, p), lambda i: (0, 0))
    full4 = pl.BlockSpec((4, p), lambda i: (0, 0))
    rowb = pl.BlockSpec((3, TILE), lambda i: (0, i))
    row4 = pl.BlockSpec((4, TILE), lambda i: (0, i))
    vecb = pl.BlockSpec((TILE,), lambda i: (i,))
    rowbf = pl.BlockSpec((TILE, 3), lambda i: (i, 0))
    fullbf = pl.BlockSpec((3, p), lambda i: (0, 0))
    maskb = pl.BlockSpec((TILE, p), lambda i: (i, 0))
    nd4b = pl.BlockSpec((TILE, 4), lambda i: (i, 0))
    c4b = pl.BlockSpec((p, 4), lambda i: (0, 0))
    c5b = pl.BlockSpec((p, 5), lambda i: (0, 0))

    c1 = jnp.concatenate([nrm.T, jnp.ones((p, 1), jnp.float32)], axis=1)
    c1hi, c1lo = _hilo(c1)

    nd4, mask = pl.pallas_call(
        _stage1_kernel,
        grid=grid,
        in_specs=[full, rowb, rowbf, fullbf, c4b, c4b],
        out_specs=[nd4b, maskb],
        out_shape=[
            jax.ShapeDtypeStruct((p, 4), jnp.float32),
            jax.ShapeDtypeStruct((p, p), jnp.int8),
        ],
        scratch_shapes=[
            pltpu.VMEM((TILE, p), jnp.float32),
            pltpu.VMEM((TILE, p), jnp.float32),
        ],
    )(pts, pts, pts_rows_bf, pts_all_bf, c1hi, c1lo)

    ndh = nd4.T
    a = jnp.sum(points[0] * nd4[:, :3], axis=1, keepdims=True)
    c2 = jnp.concatenate([nd4[:, :3], a, jnp.ones((p, 1), jnp.float32)],
                         axis=1)
    c2hi, c2lo = _hilo(c2)

    loss = pl.pallas_call(
        _stage2_kernel,
        grid=grid,
        in_specs=[full, full4, rowb, row4, maskb, c5b, c5b],
        out_specs=vecb,
        out_shape=jax.ShapeDtypeStruct((p,), jnp.float32),
    )(pts, ndh, pts, ndh, mask, c2hi, c2lo)

    return jnp.mean(loss)


def kernel(points, normals):
    return _run(points, normals)
